# baseline (speedup 1.0000x reference)
import jax
import jax.numpy as jnp
from jax.experimental import pallas as pl
from jax.experimental.pallas import tpu as pltpu

_D = 2048
_CHUNKS = [256] * 7 + [128, 128]
assert sum(_CHUNKS) == _D
_OFFS = [sum(_CHUNKS[:i]) for i in range(len(_CHUNKS))]
_NCH = len(_CHUNKS)


def _moe_kernel(x_ref, gw_ref, b_ref, w_hbm, out_ref, wbuf, obuf, sems,
                osems):
    x = x_ref[...]
    gw = gw_ref[...]
    xsum = jnp.sum(x, axis=0, keepdims=True)
    s = jax.lax.dot_general(
        xsum, gw, (((1,), (1,)), ((), ())),
        preferred_element_type=jnp.float32)
    e_iota = jax.lax.broadcasted_iota(jnp.int32, s.shape, 1)
    i0 = jnp.argmax(s, axis=1)[0]
    s_masked = jnp.where(e_iota == i0, -jnp.inf, s)
    i1 = jnp.argmax(s_masked, axis=1)[0]

    def copy(slot, idx, c):
        sl = pl.ds(_OFFS[c], _CHUNKS[c])
        return pltpu.make_async_copy(
            w_hbm.at[idx, sl, :],
            wbuf.at[slot, sl, :],
            sems.at[slot, c],
        )

    for c in range(_NCH):
        copy(0, i0, c).start()
        copy(1, i1, c).start()

    logits = jax.lax.dot_general(
        x, gw, (((1,), (1,)), ((), ())),
        preferred_element_type=jnp.float32)
    l0 = jnp.sum(jnp.where(e_iota == i0, logits, 0.0), axis=1, keepdims=True)
    l1 = jnp.sum(jnp.where(e_iota == i1, logits, 0.0), axis=1, keepdims=True)
    m = jnp.maximum(l0, l1)
    e0 = jnp.exp(l0 - m)
    e1 = jnp.exp(l1 - m)
    denom = e0 + e1
    w0 = e0 / denom
    w1 = e1 / denom

    b = b_ref[...]
    row_iota = jax.lax.broadcasted_iota(jnp.int32, b.shape, 0)
    b0 = jnp.sum(jnp.where(row_iota == i0, b, 0.0), axis=0, keepdims=True)
    b1 = jnp.sum(jnp.where(row_iota == i1, b, 0.0), axis=0, keepdims=True)
    obuf[...] = w0 * b0 + w1 * b1

    dn = (((1,), (1,)), ((), ()))

    def out_copy(c):
        sl = pl.ds(_OFFS[c], _CHUNKS[c])
        return pltpu.make_async_copy(
            obuf.at[:, sl], out_ref.at[:, sl], osems.at[c])

    for c in range(_NCH):
        sl = pl.ds(_OFFS[c], _CHUNKS[c])
        copy(0, i0, c).wait()
        copy(1, i1, c).wait()
        part0 = jax.lax.dot_general(x, wbuf[0, sl, :], dn,
                                    preferred_element_type=jnp.float32)
        part1 = jax.lax.dot_general(x, wbuf[1, sl, :], dn,
                                    preferred_element_type=jnp.float32)
        obuf[:, sl] = obuf[:, sl] + (w0 * part0 + w1 * part1)
        out_copy(c).start()

    for c in range(_NCH):
        out_copy(c).wait()


@jax.jit
def kernel(hidden_states, gate_w, expert_w, expert_b):
    B, S, d = hidden_states.shape
    T = B * S
    x = hidden_states.reshape(T, d)

    out = pl.pallas_call(
        _moe_kernel,
        in_specs=[
            pl.BlockSpec(memory_space=pltpu.MemorySpace.VMEM),
            pl.BlockSpec(memory_space=pltpu.MemorySpace.VMEM),
            pl.BlockSpec(memory_space=pltpu.MemorySpace.VMEM),
            pl.BlockSpec(memory_space=pltpu.MemorySpace.HBM),
        ],
        out_specs=pl.BlockSpec(memory_space=pltpu.MemorySpace.HBM),
        out_shape=jax.ShapeDtypeStruct((T, d), jnp.float32),
        scratch_shapes=[
            pltpu.VMEM((2, _D, _D), jnp.float32),
            pltpu.VMEM((T, _D), jnp.float32),
            pltpu.SemaphoreType.DMA((2, _NCH)),
            pltpu.SemaphoreType.DMA((_NCH,)),
        ],
    )(x, gate_w, expert_b, expert_w)

    return out.reshape(B, S, d)


# speedup vs baseline: 1.0139x; 1.0139x over previous
import jax
import jax.numpy as jnp
from jax.experimental import pallas as pl
from jax.experimental.pallas import tpu as pltpu

_CH = 256
_D = 2048
_NCH = _D // _CH


def _moe_kernel(x_ref, gw_ref, b_ref, w_hbm, out_ref, wbuf, obuf, sems,
                osems):
    x = x_ref[...]
    gw = gw_ref[...]
    xsum = jnp.sum(x, axis=0, keepdims=True)
    s = jax.lax.dot_general(
        xsum, gw, (((1,), (1,)), ((), ())),
        preferred_element_type=jnp.float32)
    e_iota = jax.lax.broadcasted_iota(jnp.int32, s.shape, 1)
    i0 = jnp.argmax(s, axis=1)[0]
    s_masked = jnp.where(e_iota == i0, -jnp.inf, s)
    i1 = jnp.argmax(s_masked, axis=1)[0]

    def copy(slot, idx, c):
        return pltpu.make_async_copy(
            w_hbm.at[idx, pl.ds(c * _CH, _CH), :],
            wbuf.at[slot, c],
            sems.at[slot, c],
        )

    def issue(c, _):
        copy(0, i0, c).start()
        copy(1, i1, c).start()
        return 0

    jax.lax.fori_loop(0, _NCH, issue, 0, unroll=True)

    logits = jax.lax.dot_general(
        x, gw, (((1,), (1,)), ((), ())),
        preferred_element_type=jnp.float32)
    l0 = jnp.sum(jnp.where(e_iota == i0, logits, 0.0), axis=1, keepdims=True)
    l1 = jnp.sum(jnp.where(e_iota == i1, logits, 0.0), axis=1, keepdims=True)
    m = jnp.maximum(l0, l1)
    e0 = jnp.exp(l0 - m)
    e1 = jnp.exp(l1 - m)
    denom = e0 + e1
    w0 = e0 / denom
    w1 = e1 / denom

    b = b_ref[...]
    row_iota = jax.lax.broadcasted_iota(jnp.int32, b.shape, 0)
    b0 = jnp.sum(jnp.where(row_iota == i0, b, 0.0), axis=0, keepdims=True)
    b1 = jnp.sum(jnp.where(row_iota == i1, b, 0.0), axis=0, keepdims=True)
    obuf[...] = w0 * b0 + w1 * b1

    dn = (((1,), (1,)), ((), ()))

    def out_copy(c):
        sl = pl.ds(c * _CH, _CH)
        return pltpu.make_async_copy(
            obuf.at[:, sl], out_ref.at[:, sl], osems.at[c])

    def compute(c, _):
        copy(0, i0, c).wait()
        copy(1, i1, c).wait()
        part0 = jax.lax.dot_general(x, wbuf[0, c], dn,
                                    preferred_element_type=jnp.float32)
        part1 = jax.lax.dot_general(x, wbuf[1, c], dn,
                                    preferred_element_type=jnp.float32)
        sl = pl.ds(c * _CH, _CH)
        obuf[:, sl] = obuf[:, sl] + (w0 * part0 + w1 * part1)
        out_copy(c).start()
        return 0

    jax.lax.fori_loop(0, _NCH, compute, 0, unroll=True)

    def drain(c, _):
        out_copy(c).wait()
        return 0

    jax.lax.fori_loop(0, _NCH, drain, 0, unroll=True)


@jax.jit
def kernel(hidden_states, gate_w, expert_w, expert_b):
    B, S, d = hidden_states.shape
    T = B * S
    x = hidden_states.reshape(T, d)

    out = pl.pallas_call(
        _moe_kernel,
        in_specs=[
            pl.BlockSpec(memory_space=pltpu.MemorySpace.VMEM),
            pl.BlockSpec(memory_space=pltpu.MemorySpace.VMEM),
            pl.BlockSpec(memory_space=pltpu.MemorySpace.VMEM),
            pl.BlockSpec(memory_space=pltpu.MemorySpace.HBM),
        ],
        out_specs=pl.BlockSpec(memory_space=pltpu.MemorySpace.HBM),
        out_shape=jax.ShapeDtypeStruct((T, d), jnp.float32),
        scratch_shapes=[
            pltpu.VMEM((2, _NCH, _CH, d), jnp.float32),
            pltpu.VMEM((T, _D), jnp.float32),
            pltpu.SemaphoreType.DMA((2, _NCH)),
            pltpu.SemaphoreType.DMA((_NCH,)),
        ],
    )(x, gate_w, expert_b, expert_w)

    return out.reshape(B, S, d)


# speedup vs baseline: 1.0143x; 1.0004x over previous
import jax
import jax.numpy as jnp
from jax.experimental import pallas as pl
from jax.experimental.pallas import tpu as pltpu

_CH = 256
_D = 2048
_NCH = _D // _CH


def _moe_kernel(x_ref, gw_ref, b_ref, w_hbm, out_ref, wbuf, obuf, sems,
                osems):
    x = x_ref[...]
    gw = gw_ref[...]
    xsum = jnp.sum(x, axis=0, keepdims=True)
    s = jax.lax.dot_general(
        xsum, gw, (((1,), (1,)), ((), ())),
        preferred_element_type=jnp.float32)
    e_iota = jax.lax.broadcasted_iota(jnp.int32, s.shape, 1)
    i0 = jnp.argmax(s, axis=1)[0]
    s_masked = jnp.where(e_iota == i0, -jnp.inf, s)
    i1 = jnp.argmax(s_masked, axis=1)[0]

    def copy(slot, idx, c):
        return pltpu.make_async_copy(
            w_hbm.at[idx, pl.ds(c * _CH, _CH), :],
            wbuf.at[slot, c],
            sems.at[slot, c],
        )

    def issue(c, _):
        copy(0, i0, c).start()
        copy(1, i1, c).start()
        return 0

    jax.lax.fori_loop(0, _NCH, issue, 0, unroll=True)

    logits = jax.lax.dot_general(
        x, gw, (((1,), (1,)), ((), ())),
        preferred_element_type=jnp.float32)
    l0 = jnp.sum(jnp.where(e_iota == i0, logits, 0.0), axis=1, keepdims=True)
    l1 = jnp.sum(jnp.where(e_iota == i1, logits, 0.0), axis=1, keepdims=True)
    m = jnp.maximum(l0, l1)
    e0 = jnp.exp(l0 - m)
    e1 = jnp.exp(l1 - m)
    denom = e0 + e1
    w0 = e0 / denom
    w1 = e1 / denom

    b = b_ref[...]
    row_iota = jax.lax.broadcasted_iota(jnp.int32, b.shape, 0)
    b0 = jnp.sum(jnp.where(row_iota == i0, b, 0.0), axis=0, keepdims=True)
    b1 = jnp.sum(jnp.where(row_iota == i1, b, 0.0), axis=0, keepdims=True)
    bias = w0 * b0 + w1 * b1

    dn = (((1,), (1,)), ((), ()))

    def out_copy(c):
        sl = pl.ds(c * _CH, _CH)
        return pltpu.make_async_copy(
            obuf.at[:, sl], out_ref.at[:, sl], osems.at[c])

    for c in range(_NCH):
        copy(0, i0, c).wait()
        copy(1, i1, c).wait()
        part0 = jax.lax.dot_general(x, wbuf[0, c], dn,
                                    preferred_element_type=jnp.float32)
        part1 = jax.lax.dot_general(x, wbuf[1, c], dn,
                                    preferred_element_type=jnp.float32)
        obuf[:, c * _CH:(c + 1) * _CH] = (
            bias[:, c * _CH:(c + 1) * _CH] + w0 * part0 + w1 * part1)
        out_copy(c).start()

    for c in range(_NCH):
        out_copy(c).wait()


@jax.jit
def kernel(hidden_states, gate_w, expert_w, expert_b):
    B, S, d = hidden_states.shape
    T = B * S
    x = hidden_states.reshape(T, d)

    out = pl.pallas_call(
        _moe_kernel,
        in_specs=[
            pl.BlockSpec(memory_space=pltpu.MemorySpace.VMEM),
            pl.BlockSpec(memory_space=pltpu.MemorySpace.VMEM),
            pl.BlockSpec(memory_space=pltpu.MemorySpace.VMEM),
            pl.BlockSpec(memory_space=pltpu.MemorySpace.HBM),
        ],
        out_specs=pl.BlockSpec(memory_space=pltpu.MemorySpace.HBM),
        out_shape=jax.ShapeDtypeStruct((T, d), jnp.float32),
        scratch_shapes=[
            pltpu.VMEM((2, _NCH, _CH, d), jnp.float32),
            pltpu.VMEM((T, _D), jnp.float32),
            pltpu.SemaphoreType.DMA((2, _NCH)),
            pltpu.SemaphoreType.DMA((_NCH,)),
        ],
    )(x, gate_w, expert_b, expert_w)

    return out.reshape(B, S, d)
